# baseline (speedup 1.0000x reference)
import dataclasses
import math
import jax
import jax.numpy as jnp
from jax import lax
from jax.experimental import pallas as pl
from jax.experimental.pallas import tpu as pltpu
from jax.experimental.pallas import tpu_sc as plsc

NC = 2
NS = 16
NW = NC * NS
LANES = 16
CHUNK = 128


def _ceil_to(a, m):
    return ((a + m - 1) // m) * m


def _sc_compiler_params():
    cp = pltpu.CompilerParams()
    if "needs_layout_passes" in pltpu.CompilerParams.__dataclass_fields__:
        cp = dataclasses.replace(cp, needs_layout_passes=False)
    return cp


def _mm1_body(x_ref, w_ref, o_ref):
    o_ref[...] = jnp.dot(x_ref[...], w_ref[...],
                         preferred_element_type=jnp.float32)


def _mm2_body(y_ref, w_ref, o_ref):
    z = jnp.maximum(y_ref[0] + y_ref[1], 0.0)
    o_ref[...] = jnp.dot(z, w_ref[...], preferred_element_type=jnp.float32)


def _mmtab_body(y_ref, w_ref, o_ref):
    z = y_ref[0] + y_ref[1]
    o_ref[...] = lax.dot_general(w_ref[...], z, (((1,), (1,)), ((), ())),
                                 preferred_element_type=jnp.float32)


CORE0_FRAC = 0.88
GCH = 128
DEPTH = 3


def _make_scatter_kernel(n, feat, nchunk0, nchunk1, npad):
    assert nchunk0 % DEPTH == 0 and nchunk1 % DEPTH == 0
    assert nchunk0 >= 2 * DEPTH and nchunk1 >= 2 * DEPTH
    zrows = _ceil_to(-(-npad // NS), 8)
    zrows_last = npad - zrows * (NS - 1)
    assert zrows_last > 0 and zrows_last % 8 == 0
    out_rows = _ceil_to(-(-n // NS), 8)
    out_rows_last = n - out_rows * (NS - 1)
    assert out_rows_last > 0 and out_rows_last % 8 == 0 and n % 8 == 0
    mesh = plsc.VectorSubcoreMesh(core_axis_name="c", subcore_axis_name="s")

    def body(h_hbm, sd_hbm, out_hbm, acc, *rest):
        sd = rest[0:DEPTH]
        rows = rest[DEPTH:2 * DEPTH]
        gsem = rest[2 * DEPTH:3 * DEPTH]
        isem = rest[3 * DEPTH:4 * DEPTH]
        c = lax.axis_index("c")
        s = lax.axis_index("s")

        @pl.loop(0, GCH)
        def _zrow(i):
            for j in range(feat // LANES):
                rows[0][pl.ds(i, 1), pl.ds(j * LANES, LANES)] = jnp.zeros(
                    (1, LANES), jnp.float32)

        def zero_span(zbase, nrows):
            for k in range(nrows // GCH):
                pltpu.sync_copy(rows[0], acc.at[pl.ds(zbase + k * GCH, GCH)])
            tail = nrows % GCH
            if tail:
                pltpu.sync_copy(
                    rows[0].at[pl.ds(0, tail)],
                    acc.at[pl.ds(zbase + (nrows // GCH) * GCH, tail)])

        @pl.when(s < NS - 1)
        def _zero_full():
            zero_span(s * zrows, zrows)

        @pl.when(s == NS - 1)
        def _zero_last():
            zero_span(zrows * (NS - 1), zrows_last)

        plsc.subcore_barrier()

        def run_edges(base, nchunk):
            def idx_start(k, b):
                pltpu.async_copy(sd_hbm.at[base + k], sd[b], isem[b])

            def idx_wait(k, b):
                pltpu.make_async_copy(sd_hbm.at[base + k], sd[b],
                                      isem[b]).wait()

            def gather_start(b):
                pltpu.async_copy(h_hbm.at[sd[b].at[0]], rows[b], gsem[b])

            def gather_wait(b):
                pltpu.make_async_copy(h_hbm.at[sd[b].at[0]], rows[b],
                                      gsem[b]).wait()

            for b in range(DEPTH):
                idx_start(b, b)
            for b in range(DEPTH - 1):
                idx_wait(b, b)
                gather_start(b)

            @pl.loop(0, nchunk // DEPTH)
            def _edge_chunk(j):
                kbase = j * DEPTH
                for b in range(DEPTH):
                    k = kbase + b
                    bn = (b + DEPTH - 1) % DEPTH

                    @pl.when(k + DEPTH - 1 < nchunk)
                    def _launch_gather():
                        idx_wait(k + DEPTH - 1, bn)
                        gather_start(bn)

                    gather_wait(b)
                    pltpu.sync_copy(rows[b], acc.at[sd[b].at[1]], add=True)

                    @pl.when(k + DEPTH < nchunk)
                    def _prefetch_idx():
                        idx_start(k + DEPTH, b)

        @pl.when(c == 0)
        def _core0():
            run_edges(s * nchunk0, nchunk0)

        @pl.when(c == 1)
        def _core1():
            run_edges(NS * nchunk0 + s * nchunk1, nchunk1)

        plsc.subcore_barrier()
        r0 = s * out_rows

        @pl.when(s < NS - 1)
        def _copy_full():
            pltpu.sync_copy(acc.at[pl.ds(r0, out_rows)],
                            out_hbm.at[c].at[pl.ds(r0, out_rows)])

        @pl.when(s == NS - 1)
        def _copy_last():
            pltpu.sync_copy(acc.at[pl.ds(out_rows * (NS - 1), out_rows_last)],
                            out_hbm.at[c].at[pl.ds(out_rows * (NS - 1),
                                                   out_rows_last)])

    return pl.kernel(
        body,
        out_type=jax.ShapeDtypeStruct((NC, n, feat), jnp.float32),
        mesh=mesh,
        scratch_types=(
            [pltpu.VMEM_SHARED((npad, feat), jnp.float32)]
            + [pltpu.VMEM((2, GCH), jnp.int32)] * DEPTH
            + [pltpu.VMEM((GCH, feat), jnp.float32)] * DEPTH
            + [pltpu.SemaphoreType.DMA] * (2 * DEPTH)
        ),
    )


def _make_decode_kernel(n, epad2, epw2):
    mesh = plsc.VectorSubcoreMesh(core_axis_name="c", subcore_axis_name="s")

    def body(tab_hbm, e0_hbm, e1_hbm, out0_hbm, out1_hbm, tab, aidx, bidx,
             out0, out1, sem):
        c = lax.axis_index("c")
        s = lax.axis_index("s")
        w = c * NS + s
        base = w * epw2
        pltpu.async_copy(tab_hbm, tab, sem).wait()
        pltpu.sync_copy(e0_hbm.at[pl.ds(base, epw2)], aidx)
        pltpu.sync_copy(e1_hbm.at[pl.ds(base, epw2)], bidx)

        r0 = jnp.zeros((LANES,), jnp.int32)
        r1 = jnp.full((LANES,), 1, jnp.int32)
        r2 = jnp.full((LANES,), 2, jnp.int32)
        r3 = jnp.full((LANES,), 3, jnp.int32)

        @pl.loop(0, epw2, step=LANES)
        def _step(p):
            i0 = aidx[pl.ds(p, LANES)]
            i1 = bidx[pl.ds(p, LANES)]
            u0 = plsc.load_gather(tab, [r0, i0])
            u1 = plsc.load_gather(tab, [r1, i0])
            v0 = plsc.load_gather(tab, [r2, i1])
            v1 = plsc.load_gather(tab, [r3, i1])
            out0[pl.ds(p, LANES)] = u0 + v0
            out1[pl.ds(p, LANES)] = u1 + v1

        pltpu.sync_copy(out0, out0_hbm.at[pl.ds(base, epw2)])
        pltpu.sync_copy(out1, out1_hbm.at[pl.ds(base, epw2)])

    return pl.kernel(
        body,
        out_type=[jax.ShapeDtypeStruct((epad2,), jnp.float32),
                  jax.ShapeDtypeStruct((epad2,), jnp.float32)],
        mesh=mesh,
        compiler_params=_sc_compiler_params(),
        scratch_types=[
            pltpu.VMEM((4, n), jnp.float32),
            pltpu.VMEM((epw2,), jnp.int32),
            pltpu.VMEM((epw2,), jnp.int32),
            pltpu.VMEM((epw2,), jnp.float32),
            pltpu.VMEM((epw2,), jnp.float32),
            pltpu.SemaphoreType.DMA,
        ],
    )


@jax.jit
def kernel(x, edge_index, pos_edge_index, neg_edge_index, W1, W2, Wlin):
    n, feat = x.shape
    hid = W1.shape[1]
    e = edge_index.shape[1]
    pe2 = pos_edge_index.shape[1] + neg_edge_index.shape[1]

    tot_chunks = -(-e // GCH)
    per_pair = -(-tot_chunks // NS)
    nchunk0 = max(2 * DEPTH, DEPTH * round(per_pair * CORE0_FRAC / DEPTH))
    nchunk1 = max(2 * DEPTH, _ceil_to(max(per_pair - nchunk0, 1), DEPTH))
    epad = NS * (nchunk0 + nchunk1) * GCH
    npad = _ceil_to(n + 1, 8)
    dummy = n

    epw2 = _ceil_to(-(-pe2 // NW), CHUNK)
    epad2 = epw2 * NW

    src = jnp.concatenate(
        [edge_index[0], jnp.zeros((epad - e,), jnp.int32)]
    ).reshape(-1, 1, GCH)
    dst = jnp.concatenate(
        [edge_index[1], jnp.full((epad - e,), dummy, jnp.int32)]
    ).reshape(-1, 1, GCH)
    srcdst = jnp.concatenate([src, dst], axis=1)
    eidx = jnp.concatenate([pos_edge_index, neg_edge_index], axis=-1)
    e0 = jnp.concatenate([eidx[0], jnp.zeros((epad2 - pe2,), jnp.int32)])
    e1 = jnp.concatenate([eidx[1], jnp.zeros((epad2 - pe2,), jnp.int32)])
    wpack = jnp.concatenate([Wlin[:, :hid], Wlin[:, hid:]], axis=0)

    h1 = pl.pallas_call(
        _mm1_body,
        out_shape=jax.ShapeDtypeStruct((n, hid), jnp.float32),
    )(x, W1)

    scat = _make_scatter_kernel(n, hid, nchunk0, nchunk1, npad)
    y1 = scat(h1, srcdst)

    h2 = pl.pallas_call(
        _mm2_body,
        out_shape=jax.ShapeDtypeStruct((n, hid), jnp.float32),
    )(y1, W2)

    y2 = scat(h2, srcdst)

    tab = pl.pallas_call(
        _mmtab_body,
        out_shape=jax.ShapeDtypeStruct((4, n), jnp.float32),
    )(y2, wpack)

    dec = _make_decode_kernel(n, epad2, epw2)
    out0, out1 = dec(tab, e0, e1)
    return jnp.stack([out0[:pe2], out1[:pe2]], axis=1)


# speedup vs baseline: 1.1243x; 1.1243x over previous
import dataclasses
import math
import jax
import jax.numpy as jnp
from jax import lax
from jax.experimental import pallas as pl
from jax.experimental.pallas import tpu as pltpu
from jax.experimental.pallas import tpu_sc as plsc

NC = 2
NS = 16
NW = NC * NS
LANES = 16
CHUNK = 128


def _ceil_to(a, m):
    return ((a + m - 1) // m) * m


def _sc_compiler_params():
    cp = pltpu.CompilerParams()
    if "needs_layout_passes" in pltpu.CompilerParams.__dataclass_fields__:
        cp = dataclasses.replace(cp, needs_layout_passes=False)
    return cp


def _mm1_body(x_ref, w_ref, o_ref):
    o_ref[...] = jnp.dot(x_ref[...], w_ref[...],
                         preferred_element_type=jnp.float32)


def _mm2_body(y_ref, w_ref, o_ref):
    z = jnp.maximum(y_ref[0] + y_ref[1], 0.0)
    o_ref[...] = jnp.dot(z, w_ref[...], preferred_element_type=jnp.float32)


def _mmtab_body(y_ref, w_ref, o_ref):
    z = y_ref[0] + y_ref[1]
    o_ref[...] = lax.dot_general(w_ref[...], z, (((1,), (1,)), ((), ())),
                                 preferred_element_type=jnp.float32)


CORE0_FRAC = 0.88


def _make_scatter_kernel(n, feat, nchunk0, nchunk1, npad):
    assert nchunk0 % 2 == 0 and nchunk1 % 2 == 0
    rows_per_tile_zero = npad // NS
    zero_reps = rows_per_tile_zero // CHUNK
    out_rows = _ceil_to(-(-n // NS), 8)
    out_rows_last = n - out_rows * (NS - 1)
    assert out_rows_last > 0 and out_rows_last % 8 == 0 and n % 8 == 0
    mesh = plsc.VectorSubcoreMesh(core_axis_name="c", subcore_axis_name="s")

    def body(h_hbm, src_hbm, dst_hbm, out_hbm, acc, sidx0, sidx1, didx0,
             didx1, rows0, rows1, gsem0, gsem1, isem0, isem1):
        c = lax.axis_index("c")
        s = lax.axis_index("s")

        @pl.loop(0, CHUNK)
        def _zrow(i):
            for j in range(feat // LANES):
                rows0[pl.ds(i, 1), pl.ds(j * LANES, LANES)] = jnp.zeros(
                    (1, LANES), jnp.float32)

        zbase = s * rows_per_tile_zero
        for k in range(zero_reps):
            pltpu.sync_copy(rows0, acc.at[pl.ds(zbase + k * CHUNK, CHUNK)])
        plsc.subcore_barrier()

        def run_edges(base, nchunk, prio):
            def idx_start(k, sbuf, dbuf, sem):
                pltpu.async_copy(src_hbm.at[base + k], sbuf, sem)
                pltpu.async_copy(dst_hbm.at[base + k], dbuf, sem)

            def idx_wait(k, sbuf, dbuf, sem):
                pltpu.make_async_copy(src_hbm.at[base + k], sbuf, sem).wait()
                pltpu.make_async_copy(dst_hbm.at[base + k], dbuf, sem).wait()

            pltpu.sync_copy(src_hbm.at[base], sidx0)
            pltpu.sync_copy(dst_hbm.at[base], didx0)
            pltpu.async_copy(h_hbm.at[sidx0], rows0, gsem0, priority=prio)
            idx_start(1, sidx1, didx1, isem1)

            @pl.loop(0, nchunk // 2 - 1)
            def _edge_chunk(j):
                k = 2 * j
                idx_wait(k + 1, sidx1, didx1, isem1)
                pltpu.async_copy(h_hbm.at[sidx1], rows1, gsem1, priority=prio)
                pltpu.make_async_copy(h_hbm.at[sidx0], rows0, gsem0).wait()
                pltpu.async_copy(src_hbm.at[base + k + 2], sidx0, isem0)
                pltpu.sync_copy(rows0, acc.at[didx0], add=True)
                pltpu.async_copy(dst_hbm.at[base + k + 2], didx0, isem0)
                idx_wait(k + 2, sidx0, didx0, isem0)
                pltpu.async_copy(h_hbm.at[sidx0], rows0, gsem0, priority=prio)
                pltpu.make_async_copy(h_hbm.at[sidx1], rows1, gsem1).wait()
                pltpu.async_copy(src_hbm.at[base + k + 3], sidx1, isem1)
                pltpu.sync_copy(rows1, acc.at[didx1], add=True)
                pltpu.async_copy(dst_hbm.at[base + k + 3], didx1, isem1)

            last = nchunk - 2
            idx_wait(last + 1, sidx1, didx1, isem1)
            pltpu.async_copy(h_hbm.at[sidx1], rows1, gsem1, priority=prio)
            pltpu.make_async_copy(h_hbm.at[sidx0], rows0, gsem0).wait()
            pltpu.sync_copy(rows0, acc.at[didx0], add=True)
            pltpu.make_async_copy(h_hbm.at[sidx1], rows1, gsem1).wait()
            pltpu.sync_copy(rows1, acc.at[didx1], add=True)

        @pl.when(c == 0)
        def _core0():
            run_edges(s * nchunk0, nchunk0, 0)

        @pl.when(c == 1)
        def _core1():
            run_edges(NS * nchunk0 + s * nchunk1, nchunk1, 1)

        plsc.subcore_barrier()
        r0 = s * out_rows

        @pl.when(s < NS - 1)
        def _copy_full():
            pltpu.sync_copy(acc.at[pl.ds(r0, out_rows)],
                            out_hbm.at[c].at[pl.ds(r0, out_rows)])

        @pl.when(s == NS - 1)
        def _copy_last():
            pltpu.sync_copy(acc.at[pl.ds(out_rows * (NS - 1), out_rows_last)],
                            out_hbm.at[c].at[pl.ds(out_rows * (NS - 1),
                                                   out_rows_last)])

    return pl.kernel(
        body,
        out_type=jax.ShapeDtypeStruct((NC, n, feat), jnp.float32),
        mesh=mesh,
        scratch_types=[
            pltpu.VMEM_SHARED((npad, feat), jnp.float32),
            pltpu.VMEM((CHUNK,), jnp.int32),
            pltpu.VMEM((CHUNK,), jnp.int32),
            pltpu.VMEM((CHUNK,), jnp.int32),
            pltpu.VMEM((CHUNK,), jnp.int32),
            pltpu.VMEM((CHUNK, feat), jnp.float32),
            pltpu.VMEM((CHUNK, feat), jnp.float32),
            pltpu.SemaphoreType.DMA,
            pltpu.SemaphoreType.DMA,
            pltpu.SemaphoreType.DMA,
            pltpu.SemaphoreType.DMA,
        ],
    )


def _make_decode_kernel(n, epad2, epw2):
    mesh = plsc.VectorSubcoreMesh(core_axis_name="c", subcore_axis_name="s")

    def body(tab_hbm, e0_hbm, e1_hbm, out0_hbm, out1_hbm, tab, aidx, bidx,
             out0, out1, sem):
        c = lax.axis_index("c")
        s = lax.axis_index("s")
        w = c * NS + s
        base = w * epw2
        pltpu.async_copy(tab_hbm, tab, sem).wait()
        pltpu.sync_copy(e0_hbm.at[pl.ds(base, epw2)], aidx)
        pltpu.sync_copy(e1_hbm.at[pl.ds(base, epw2)], bidx)

        r0 = jnp.zeros((LANES,), jnp.int32)
        r1 = jnp.full((LANES,), 1, jnp.int32)
        r2 = jnp.full((LANES,), 2, jnp.int32)
        r3 = jnp.full((LANES,), 3, jnp.int32)

        @pl.loop(0, epw2, step=LANES)
        def _step(p):
            i0 = aidx[pl.ds(p, LANES)]
            i1 = bidx[pl.ds(p, LANES)]
            u0 = plsc.load_gather(tab, [r0, i0])
            u1 = plsc.load_gather(tab, [r1, i0])
            v0 = plsc.load_gather(tab, [r2, i1])
            v1 = plsc.load_gather(tab, [r3, i1])
            out0[pl.ds(p, LANES)] = u0 + v0
            out1[pl.ds(p, LANES)] = u1 + v1

        pltpu.sync_copy(out0, out0_hbm.at[pl.ds(base, epw2)])
        pltpu.sync_copy(out1, out1_hbm.at[pl.ds(base, epw2)])

    return pl.kernel(
        body,
        out_type=[jax.ShapeDtypeStruct((epad2,), jnp.float32),
                  jax.ShapeDtypeStruct((epad2,), jnp.float32)],
        mesh=mesh,
        compiler_params=_sc_compiler_params(),
        scratch_types=[
            pltpu.VMEM((4, n), jnp.float32),
            pltpu.VMEM((epw2,), jnp.int32),
            pltpu.VMEM((epw2,), jnp.int32),
            pltpu.VMEM((epw2,), jnp.float32),
            pltpu.VMEM((epw2,), jnp.float32),
            pltpu.SemaphoreType.DMA,
        ],
    )


@jax.jit
def kernel(x, edge_index, pos_edge_index, neg_edge_index, W1, W2, Wlin):
    n, feat = x.shape
    hid = W1.shape[1]
    e = edge_index.shape[1]
    pe2 = pos_edge_index.shape[1] + neg_edge_index.shape[1]

    tot_chunks = -(-e // CHUNK)
    per_pair = -(-tot_chunks // NS)
    nchunk0 = max(2, 2 * round(per_pair * CORE0_FRAC / 2))
    nchunk1 = max(2, _ceil_to(per_pair - nchunk0, 2))
    epad = NS * (nchunk0 + nchunk1) * CHUNK
    npad = _ceil_to(n + 1, NS * CHUNK)
    dummy = n

    epw2 = _ceil_to(-(-pe2 // NW), CHUNK)
    epad2 = epw2 * NW

    src = jnp.concatenate(
        [edge_index[0], jnp.zeros((epad - e,), jnp.int32)]
    ).reshape(-1, CHUNK)
    dst = jnp.concatenate(
        [edge_index[1], jnp.full((epad - e,), dummy, jnp.int32)]
    ).reshape(-1, CHUNK)
    eidx = jnp.concatenate([pos_edge_index, neg_edge_index], axis=-1)
    e0 = jnp.concatenate([eidx[0], jnp.zeros((epad2 - pe2,), jnp.int32)])
    e1 = jnp.concatenate([eidx[1], jnp.zeros((epad2 - pe2,), jnp.int32)])
    wpack = jnp.concatenate([Wlin[:, :hid], Wlin[:, hid:]], axis=0)

    h1 = pl.pallas_call(
        _mm1_body,
        out_shape=jax.ShapeDtypeStruct((n, hid), jnp.float32),
    )(x, W1)

    scat = _make_scatter_kernel(n, hid, nchunk0, nchunk1, npad)
    y1 = scat(h1, src, dst)

    h2 = pl.pallas_call(
        _mm2_body,
        out_shape=jax.ShapeDtypeStruct((n, hid), jnp.float32),
    )(y1, W2)

    y2 = scat(h2, src, dst)

    tab = pl.pallas_call(
        _mmtab_body,
        out_shape=jax.ShapeDtypeStruct((4, n), jnp.float32),
    )(y2, wpack)

    dec = _make_decode_kernel(n, epad2, epw2)
    out0, out1 = dec(tab, e0, e1)
    return jnp.stack([out0[:pe2], out1[:pe2]], axis=1)
